# baseline (speedup 1.0000x reference)
import functools

import jax
import jax.numpy as jnp
from jax import lax
from jax.experimental import pallas as pl
from jax.experimental.pallas import tpu as pltpu
from jax.experimental.pallas import tpu_sc as plsc

_EPS = 1e-5
_CHUNK = 128
_GRP = 16 * _CHUNK
_ZR = 64
_LEVELS = [313, 625, 1250, 2500, 5000, 10000]
_PREC = None


def _ceil(a, b):
    return -(-a // b)


def _r8(n):
    return _ceil(n, 8) * 8


@functools.cache
def _segsum_fn(n_in, n_out, e_pad, half):
    assert e_pad % _GRP == 0
    epw = e_pad // 16
    nch = epw // _CHUNK
    rows_pt = _ceil(n_out, 128) * 8
    nz = _ceil(n_out + 1, 16 * _ZR)
    n_acc = 16 * nz * _ZR
    npack = half // 16
    mesh = plsc.VectorSubcoreMesh(core_axis_name="c", subcore_axis_name="s")

    def body(x0, x1, src, dst, out0, out1, idx_s, idx_d, rowbuf, zbuf, acc, sem):
        c = lax.axis_index("c")
        s = lax.axis_index("s")

        def zb(i, _):
            r = i // npack
            k = i % npack
            zbuf[r, pl.ds(k * 16, 16)] = jnp.zeros((16,), jnp.float32)
            return 0
        lax.fori_loop(0, _ZR * npack, zb, 0)
        for k in range(nz):
            zo = pl.multiple_of((s * nz + k) * _ZR, _ZR)
            pltpu.sync_copy(zbuf, acc.at[pl.ds(zo, _ZR)])
        plsc.subcore_barrier()

        def run(x_ref, out_ref):
            def step(j, _):
                off = pl.multiple_of((s * nch + j) * _CHUNK, _CHUNK)
                pltpu.sync_copy(src.at[pl.ds(off, _CHUNK)], idx_s)
                pltpu.sync_copy(dst.at[pl.ds(off, _CHUNK)], idx_d)
                pltpu.async_copy(x_ref.at[idx_s], rowbuf, sem).wait()
                pltpu.sync_copy(rowbuf, acc.at[idx_d], add=True)
                return 0
            lax.fori_loop(0, nch, step, 0)
            plsc.subcore_barrier()
            for sv in range(16):
                start = sv * rows_pt
                size = min(rows_pt, _r8(n_out - start))
                if n_out - start <= 0:
                    break

                @pl.when(s == sv)
                def _(start=start, size=size):
                    pltpu.sync_copy(acc.at[pl.ds(start, size)],
                                    out_ref.at[pl.ds(start, size)])

        @pl.when(c == 0)
        def _():
            run(x0, out0)

        @pl.when(c == 1)
        def _():
            run(x1, out1)

    return pl.kernel(
        body, mesh=mesh,
        out_type=[jax.ShapeDtypeStruct((16 * rows_pt, half), jnp.float32)] * 2,
        scratch_types=[
            pltpu.VMEM((_CHUNK,), jnp.int32),
            pltpu.VMEM((_CHUNK,), jnp.int32),
            pltpu.VMEM((_CHUNK, half), jnp.float32),
            pltpu.VMEM((_ZR, half), jnp.float32),
            pltpu.VMEM_SHARED((n_acc, half), jnp.float32),
            pltpu.SemaphoreType.DMA,
        ])


@functools.cache
def _count_fn(n_out, e_pad2):
    assert e_pad2 % (2 * _GRP) == 0
    e2 = e_pad2 // 2
    ept = e2 // 16
    nch = ept // _CHUNK
    rows_pt = _ceil(n_out, 128) * 8
    nz = _ceil(n_out + 1, 16 * _ZR)
    n_acc = 16 * nz * _ZR
    mesh = plsc.VectorSubcoreMesh(core_axis_name="c", subcore_axis_name="s")

    def body(dst, out0, out1, idx_d, onesbuf, zbuf, acc, sem):
        c = lax.axis_index("c")
        s = lax.axis_index("s")

        def fill(i, _):
            r = i // 8
            k = i % 8
            zbuf[r, pl.ds(k * 16, 16)] = jnp.zeros((16,), jnp.float32)
            ob_r = i // 8
            onesbuf[ob_r, pl.ds(k * 16, 16)] = jnp.ones((16,), jnp.float32)
            return 0
        lax.fori_loop(0, _ZR * 8, fill, 0)

        def fill2(i, _):
            r = _ZR + i // 8
            k = i % 8
            onesbuf[r, pl.ds(k * 16, 16)] = jnp.ones((16,), jnp.float32)
            return 0
        lax.fori_loop(0, (_CHUNK - _ZR) * 8, fill2, 0)
        for k in range(nz):
            zo = pl.multiple_of((s * nz + k) * _ZR, _ZR)
            pltpu.sync_copy(zbuf, acc.at[pl.ds(zo, _ZR)])
        plsc.subcore_barrier()

        def step(j, _):
            off = pl.multiple_of(c * e2 + s * ept + j * _CHUNK, _CHUNK)
            pltpu.sync_copy(dst.at[pl.ds(off, _CHUNK)], idx_d)
            pltpu.sync_copy(onesbuf, acc.at[idx_d], add=True)
            return 0
        lax.fori_loop(0, nch, step, 0)
        plsc.subcore_barrier()

        def copy_out(out_ref):
            for sv in range(16):
                start = sv * rows_pt
                size = min(rows_pt, _r8(n_out - start))
                if n_out - start <= 0:
                    break

                @pl.when(s == sv)
                def _(start=start, size=size):
                    pltpu.sync_copy(acc.at[pl.ds(start, size)],
                                    out_ref.at[pl.ds(start, size)])

        @pl.when(c == 0)
        def _():
            copy_out(out0)

        @pl.when(c == 1)
        def _():
            copy_out(out1)

    return pl.kernel(
        body, mesh=mesh,
        out_type=[jax.ShapeDtypeStruct((16 * rows_pt, 128), jnp.float32)] * 2,
        scratch_types=[
            pltpu.VMEM((_CHUNK,), jnp.int32),
            pltpu.VMEM((_CHUNK, 128), jnp.float32),
            pltpu.VMEM((_ZR, 128), jnp.float32),
            pltpu.VMEM_SHARED((n_acc, 128), jnp.float32),
            pltpu.SemaphoreType.DMA,
        ])


@functools.cache
def _gather_full_fn(n_tab, b_pad, width):
    assert b_pad % (2 * _GRP) == 0 and width % 128 == 0
    b2 = b_pad // 2
    bpt = b2 // 16
    nch = bpt // _CHUNK
    mesh = plsc.VectorSubcoreMesh(core_axis_name="c", subcore_axis_name="s")

    def body(tab, idx, out, idxbuf, rowbuf, sem):
        c = lax.axis_index("c")
        s = lax.axis_index("s")

        def step(j, _):
            off = pl.multiple_of(c * b2 + s * bpt + j * _CHUNK, _CHUNK)
            pltpu.sync_copy(idx.at[pl.ds(off, _CHUNK)], idxbuf)
            pltpu.async_copy(tab.at[idxbuf], rowbuf, sem).wait()
            pltpu.sync_copy(rowbuf, out.at[pl.ds(off, _CHUNK)])
            return 0
        lax.fori_loop(0, nch, step, 0)

    return pl.kernel(
        body, mesh=mesh,
        out_type=jax.ShapeDtypeStruct((b_pad, width), jnp.float32),
        scratch_types=[
            pltpu.VMEM((_CHUNK,), jnp.int32),
            pltpu.VMEM((_CHUNK, width), jnp.float32),
            pltpu.SemaphoreType.DMA,
        ])


def _dot(a, b):
    return jax.lax.dot_general(a, b, (((1,), (0,)), ((), ())),
                               preferred_element_type=jnp.float32,
                               precision=_PREC)


_RB = 1024


def _rows_spec(w):
    return pl.BlockSpec((None, w), lambda i: (i, 0))


def _fixed_spec(r, c):
    return pl.BlockSpec((r, c), lambda i: (0, 0))


def _lin_stats(xs, n, W, bias=None, agg=None, Wn=None, deg=None):
    cout = W.shape[1]
    half = cout // 2
    has_agg = agg is not None
    widths = [x.shape[1] for x in xs]
    R = _RB if n > _RB else _r8(n)
    grid = _ceil(n, R)

    inputs = list(xs)
    specs = [pl.BlockSpec((R, w), lambda i: (i, 0)) for w in widths]
    if has_agg:
        awidths = [a.shape[1] for a in agg]
        inputs += list(agg) + [deg]
        specs += [pl.BlockSpec((R, w), lambda i: (i, 0)) for w in awidths]
        specs += [pl.BlockSpec((R, 1), lambda i: (i, 0))]
        inputs.append(Wn)
        specs.append(_fixed_spec(Wn.shape[0], cout))
    inputs.append(W)
    specs.append(_fixed_spec(W.shape[0], cout))
    if bias is not None:
        inputs.append(bias.reshape(1, cout))
        specs.append(_fixed_spec(1, cout))

    def body(*refs):
        ins = refs[:len(inputs)]
        y0, y1, s1 = refs[len(inputs):]
        i = pl.program_id(0)
        p = 0
        xv = [ins[p + k][...] for k in range(len(widths))]
        p += len(widths)
        if has_agg:
            av = [ins[p + k][...] for k in range(len(awidths))]
            p += len(awidths)
            degv = ins[p][...]
            Wnv = ins[p + 1][...]
            p += 2
        Wv = ins[p][...]
        p += 1
        xcat = xv[0] if len(xv) == 1 else jnp.concatenate(xv, axis=1)
        y = _dot(xcat, Wv)
        if has_agg:
            acat = av[0] if len(av) == 1 else jnp.concatenate(av, axis=1)
            acat = acat / jnp.maximum(degv, 1.0)
            y = y + _dot(acat, Wnv)
        if bias is not None:
            y = y + ins[p][...]
        rowid = i * R + jax.lax.broadcasted_iota(jnp.int32, (R, 1), 0)
        ym = jnp.where(rowid < n, y, 0.0)

        @pl.when(i == 0)
        def _():
            s1[...] = jnp.zeros((1, cout), jnp.float32)
        s1[...] += jnp.sum(ym, axis=0, keepdims=True)
        y0[...] = y[:, :half]
        y1[...] = y[:, half:]

    outs = pl.pallas_call(
        body,
        grid=(grid,),
        in_specs=specs,
        out_specs=[pl.BlockSpec((R, half), lambda i: (i, 0))] * 2 +
                  [_fixed_spec(1, cout)],
        out_shape=[jax.ShapeDtypeStruct((_r8(n), half), jnp.float32)] * 2 +
                  [jax.ShapeDtypeStruct((1, cout), jnp.float32)],
    )(*inputs)
    return (outs[0], outs[1]), outs[2]


def _var_pass(y, n, s1):
    half = y[0].shape[1]
    cout = 2 * half
    R = _RB if n > _RB else _r8(n)
    grid = _ceil(n, R)
    inv_n = 1.0 / n

    def body(y0, y1, s1r, s2):
        i = pl.program_id(0)
        m = s1r[...] * inv_n
        yv = jnp.concatenate([y0[...], y1[...]], axis=1)
        d = yv - m
        rowid = i * R + jax.lax.broadcasted_iota(jnp.int32, (R, 1), 0)
        d = jnp.where(rowid < n, d, 0.0)

        @pl.when(i == 0)
        def _():
            s2[...] = jnp.zeros((1, cout), jnp.float32)
        s2[...] += jnp.sum(d * d, axis=0, keepdims=True)

    return pl.pallas_call(
        body,
        grid=(grid,),
        in_specs=[pl.BlockSpec((R, half), lambda i: (i, 0))] * 2 +
                 [_fixed_spec(1, cout)],
        out_specs=_fixed_spec(1, cout),
        out_shape=jax.ShapeDtypeStruct((1, cout), jnp.float32),
    )(y[0], y[1], s1)


def _bn_apply(y, n, s1, s2, g, be, res=None, res_bn=None, relu=True):
    half = y[0].shape[1]
    cout = 2 * half
    has_res = res is not None
    has_rbn = res_bn is not None
    R = _RB if n > _RB else _r8(n)
    grid = _ceil(n, R)

    inputs = [y[0], y[1], s1, s2, g.reshape(1, cout), be.reshape(1, cout)]
    specs = [pl.BlockSpec((R, half), lambda i: (i, 0))] * 2 + \
            [_fixed_spec(1, cout)] * 4
    if has_res:
        inputs += [res[0], res[1]]
        specs += [pl.BlockSpec((R, half), lambda i: (i, 0))] * 2
    if has_rbn:
        rp, rs1, rs2, rg, rbe = res_bn
        inputs += [rp[0], rp[1], rs1, rs2,
                   rg.reshape(1, cout), rbe.reshape(1, cout)]
        specs += [pl.BlockSpec((R, half), lambda i: (i, 0))] * 2 + \
                 [_fixed_spec(1, cout)] * 4
    inv_n = 1.0 / n

    def body(*refs):
        ins = refs[:len(inputs)]
        o0, o1 = refs[len(inputs):]
        yv = jnp.concatenate([ins[0][...], ins[1][...]], axis=1)
        m = ins[2][...] * inv_n
        var = ins[3][...] * inv_n
        gv = ins[4][...]
        bev = ins[5][...]
        p = 6
        out = gv * (yv - m) * lax.rsqrt(var + _EPS) + bev
        if has_res:
            out = out + jnp.concatenate([ins[p][...], ins[p + 1][...]], axis=1)
            p += 2
        if has_rbn:
            rv = jnp.concatenate([ins[p][...], ins[p + 1][...]], axis=1)
            rm = ins[p + 2][...] * inv_n
            rvar = ins[p + 3][...] * inv_n
            rgv = ins[p + 4][...]
            rbev = ins[p + 5][...]
            out = out + (rgv * (rv - rm) * lax.rsqrt(rvar + _EPS) + rbev)
        if relu:
            out = jnp.maximum(out, 0.0)
        o0[...] = out[:, :half]
        o1[...] = out[:, half:]

    return pl.pallas_call(
        body,
        grid=(grid,),
        in_specs=specs,
        out_specs=[pl.BlockSpec((R, half), lambda i: (i, 0))] * 2,
        out_shape=[jax.ShapeDtypeStruct((_r8(n), half), jnp.float32)] * 2,
    )(*inputs)


def _convbn(xs, n, Ws, b, g, be, agg=None, Wn=None, deg=None,
            res=None, res_bn=None, relu=True):
    y, s1 = _lin_stats(xs, n, Ws, bias=b, agg=agg, Wn=Wn, deg=deg)
    s2 = _var_pass(y, n, s1)
    return _bn_apply(y, n, s1, s2, g, be, res=res, res_bn=res_bn, relu=relu)


def _linear(xs, n, W, bias=None, halves=2):
    cout = W.shape[1]
    half = cout // halves
    widths = [x.shape[1] for x in xs]
    R = _RB if n > _RB else _r8(n)
    grid = _ceil(n, R)
    inputs = list(xs) + [W]
    specs = [pl.BlockSpec((R, w), lambda i: (i, 0)) for w in widths]
    specs.append(_fixed_spec(W.shape[0], cout))
    if bias is not None:
        inputs.append(bias.reshape(1, cout))
        specs.append(_fixed_spec(1, cout))

    def body(*refs):
        ins = refs[:len(inputs)]
        outs = refs[len(inputs):]
        Wv = ins[len(widths)][...]
        y = jnp.zeros((R, cout), jnp.float32)
        off = 0
        for p, w in enumerate(widths):
            y = y + _dot(ins[p][...], Wv[off:off + w, :])
            off += w
        if bias is not None:
            y = y + ins[len(widths) + 1][...]
        for h, o in enumerate(outs):
            o[...] = y[:, h * half:(h + 1) * half]

    out = pl.pallas_call(
        body,
        grid=(grid,),
        in_specs=specs,
        out_specs=[pl.BlockSpec((R, half), lambda i: (i, 0))] * halves,
        out_shape=[jax.ShapeDtypeStruct((_r8(n), half), jnp.float32)] * halves,
    )(*inputs)
    return out[0] if halves == 1 else out


def _rowscale(pair, cnt, n):
    h = pair[0].shape[1]
    R = _RB if n > _RB else _r8(n)
    grid = _ceil(n, R)

    def body(a0, a1, c, o0, o1):
        cv = jnp.maximum(c[...], 1.0)
        o0[...] = a0[...] / cv
        o1[...] = a1[...] / cv

    return pl.pallas_call(
        body,
        grid=(grid,),
        in_specs=[pl.BlockSpec((R, h), lambda i: (i, 0))] * 2 +
                 [pl.BlockSpec((R, 1), lambda i: (i, 0))],
        out_specs=[pl.BlockSpec((R, h), lambda i: (i, 0))] * 2,
        out_shape=[jax.ShapeDtypeStruct((_r8(n), h), jnp.float32)] * 2,
    )(pair[0], pair[1], cnt)


def _mlp(ei, ej, m):
    b_pad = ei.shape[0]
    RB = 2048
    grid = (b_pad // RB,)
    W3p = jnp.pad(m['W3'], ((0, 0), (0, 127)))
    b3p = jnp.pad(m['b3'].reshape(1, 1), ((0, 0), (0, 127)))

    def body(eir, ejr, W1, b1, W2, b2, W3, b3, out):
        d = eir[...] - ejr[...]
        d = d * d
        h1 = jnp.maximum(_dot(d, W1[...]) + b1[...], 0.0)
        h2 = jnp.maximum(_dot(h1, W2[...]) + b2[...], 0.0)
        out[...] = _dot(h2, W3[...]) + b3[...]

    rows = lambda i: (i, 0)
    fixed = lambda i: (0, 0)
    out = pl.pallas_call(
        body,
        grid=grid,
        in_specs=[
            pl.BlockSpec((RB, 128), rows), pl.BlockSpec((RB, 128), rows),
            pl.BlockSpec((128, 128), fixed), pl.BlockSpec((1, 128), fixed),
            pl.BlockSpec((128, 128), fixed), pl.BlockSpec((1, 128), fixed),
            pl.BlockSpec((128, 128), fixed), pl.BlockSpec((1, 128), fixed),
        ],
        out_specs=pl.BlockSpec((RB, 128), rows),
        out_shape=jax.ShapeDtypeStruct((b_pad, 128), jnp.float32),
    )(ei, ej, m['W1'], m['b1'].reshape(1, 128),
      m['W2'], m['b2'].reshape(1, 128), W3p, b3p)
    return out


def _pad_idx(idx, pad_val, grp=_GRP):
    e = idx.shape[0]
    e_pad = _ceil(e, grp) * grp
    if e_pad != e:
        idx = jnp.concatenate(
            [idx, jnp.full((e_pad - e,), pad_val, jnp.int32)])
    return idx, e_pad


def _counts(n_out, dst):
    dst2, e_pad2 = _pad_idx(dst, n_out, 2 * _GRP)
    c0, c1 = _count_fn(n_out, e_pad2)(dst2)
    return (c0 + c1)[:, :1]


def _segsum2(x0, x1, src_pad, e_real, dst_sorted, n_out):
    gf = _gather_full_fn(x0.shape[0], src_pad.shape[0], 128)
    m0 = gf(x0, src_pad)
    m1 = gf(x1, src_pad)
    a0 = jax.ops.segment_sum(m0[:e_real], dst_sorted, num_segments=n_out)
    a1 = jax.ops.segment_sum(m1[:e_real], dst_sorted, num_segments=n_out)
    pad = _r8(n_out) - n_out
    return (jnp.pad(a0, ((0, pad), (0, 0))),
            jnp.pad(a1, ((0, pad), (0, 0))))


def _resblock(x, n, src_pad, e_real, dst_sorted, deg, p):
    a1 = _segsum2(x[0], x[1], src_pad, e_real, dst_sorted, n)
    c1 = p['c1']
    h1 = _convbn(list(x), n, c1['Ws'], c1['b'], c1['g'], c1['be'],
                 agg=list(a1), Wn=c1['Wn'], deg=deg, relu=True)
    a2 = _segsum2(h1[0], h1[1], src_pad, e_real, dst_sorted, n)
    c2 = p['c2']
    return _convbn(list(h1), n, c2['Ws'], c2['b'], c2['g'], c2['be'],
                   agg=list(a2), Wn=c2['Wn'], deg=deg, res=x, relu=True)


def _resblock512(xs4, n, src_pad, e_real, dst_sorted, deg, p):
    c1 = p['c1']
    a01 = _segsum2(xs4[0], xs4[1], src_pad, e_real, dst_sorted, n)
    a23 = _segsum2(xs4[2], xs4[3], src_pad, e_real, dst_sorted, n)
    h1 = _convbn(xs4, n, c1['Ws'], c1['b'], c1['g'], c1['be'],
                 agg=[a01[0], a01[1], a23[0], a23[1]], Wn=c1['Wn'],
                 deg=deg, relu=True)
    a2 = _segsum2(h1[0], h1[1], src_pad, e_real, dst_sorted, n)
    c2 = p['c2']
    sk_y, sk1 = _lin_stats(xs4, n, p['skip']['W'])
    sk2 = _var_pass(sk_y, n, sk1)
    return _convbn(list(h1), n, c2['Ws'], c2['b'], c2['g'], c2['be'],
                   agg=list(a2), Wn=c2['Wn'], deg=deg,
                   res_bn=(sk_y, sk1, sk2, p['skip']['g'], p['skip']['be']),
                   relu=True)


def kernel(data, edge_index_0, edge_index_1, edge_index_2, edge_index_3,
           edge_index_4, edge_index_5, cluster_1, cluster_2, cluster_3,
           cluster_4, cluster_5, dist, depth, params):
    edges = [edge_index_0, edge_index_1, edge_index_2, edge_index_3,
             edge_index_4, edge_index_5]
    clusters = {1: cluster_1, 2: cluster_2, 3: cluster_3, 4: cluster_4,
                5: cluster_5}

    srcs, dsts, ereal, degs = {}, {}, {}, {}
    for l in range(6):
        n = _LEVELS[l]
        perm = jnp.argsort(edges[l][1], stable=True).astype(jnp.int32)
        s, _ = _pad_idx(edges[l][0][perm], 0, 2 * _GRP)
        srcs[l] = s
        dsts[l] = edges[l][1][perm]
        ereal[l] = edges[l].shape[1]
        degs[l] = _counts(n, edges[l][1])

    n5 = _LEVELS[5]
    p1 = params['conv1']
    gf5 = _gather_full_fn(n5, srcs[5].shape[0], 128)
    m5 = gf5(data, srcs[5])
    a_full = jax.ops.segment_sum(m5[:ereal[5]], dsts[5], num_segments=n5)
    x = _convbn([data], n5, p1['Ws'], p1['b'], p1['g'], p1['be'],
                agg=[a_full], Wn=p1['Wn'], deg=degs[5], relu=True)
    convd = {5: x}

    for i in range(5):
        d = 5 - i
        n_in, n_out = _LEVELS[d], _LEVELS[d - 1]
        cl = clusters[d]
        perm = jnp.argsort(cl, stable=True).astype(jnp.int32)
        iota, _ = _pad_idx(perm, 0, 2 * _GRP)
        s_pair = _segsum2(convd[d][0], convd[d][1], iota, n_in,
                          cl[perm], n_out)
        cnt = _counts(n_out, cl)
        x = _rowscale(s_pair, cnt, n_out)
        for bp in params['enc'][i]:
            x = _resblock(x, n_out, srcs[d - 1], ereal[d - 1],
                          dsts[d - 1], degs[d - 1], bp)
        convd[d - 1] = x

    deconv = convd[0]
    for i in range(5):
        lvl = i + 1
        n = _LEVELS[lvl]
        cidx, bpad = _pad_idx(clusters[lvl], 0, 2 * _GRP)
        gf = _gather_full_fn(_LEVELS[lvl - 1], bpad, 128)
        up = (gf(deconv[0], cidx), gf(deconv[1], cidx))
        xs4 = [convd[lvl][0], convd[lvl][1], up[0], up[1]]
        for j, bp in enumerate(params['dec'][i]):
            if j == 0:
                deconv = _resblock512(xs4, n, srcs[lvl], ereal[lvl],
                                      dsts[lvl], degs[lvl], bp)
            else:
                deconv = _resblock(deconv, n, srcs[lvl], ereal[lvl],
                                   dsts[lvl], degs[lvl], bp)

    hd = params['header']
    hh = _convbn(list(deconv), n5, hd['W1'], None, hd['g1'], hd['b1'],
                 relu=True)
    emb = _linear(list(hh), n5, hd['W2'], bias=hd['bias2'], halves=1)

    ii, bp2 = _pad_idx(dist[:, 0], 0, 2 * _GRP)
    jj, _ = _pad_idx(dist[:, 1], 0, 2 * _GRP)
    ei = _gather_full_fn(n5, bp2, 128)(emb, ii)
    ej = _gather_full_fn(n5, bp2, 128)(emb, jj)
    out = _mlp(ei, ej, params['mlp'])
    return out[:dist.shape[0], 0]
